# baseline (speedup 1.0000x reference)
import jax
import jax.numpy as jnp
from jax import lax
from jax.experimental import pallas as pl
from jax.experimental.pallas import tpu as pltpu
from jax.experimental.pallas import tpu_sc as plsc

BATCH = 16384
EMBED_DIM = 64
ROW = 2 * EMBED_DIM
NUM_CORES = 2
NUM_SUBCORES = 16
NUM_TILES = NUM_CORES * NUM_SUBCORES
B_PER_TILE = BATCH // NUM_TILES
CHUNK = 128
NCHUNK = B_PER_TILE // CHUNK

_MESH = plsc.VectorSubcoreMesh(core_axis_name="c", subcore_axis_name="s")


def _gather_body(uidx_hbm, iidx_hbm, uemb_hbm, iemb_hbm, u_hbm, v_hbm,
                 uidx_v, iidx_v, urows_v, irows_v, sem):
    wid = lax.axis_index("s") * NUM_CORES + lax.axis_index("c")
    base = wid * B_PER_TILE
    pltpu.sync_copy(uidx_hbm.at[wid], uidx_v)
    pltpu.sync_copy(iidx_hbm.at[wid], iidx_v)

    def start(j):
        b = j % 2
        return (
            pltpu.async_copy(uemb_hbm.at[uidx_v.at[j]], urows_v.at[b], sem),
            pltpu.async_copy(iemb_hbm.at[iidx_v.at[j]], irows_v.at[b], sem),
        )

    pend = start(0)
    for j in range(NCHUNK):
        nxt = start(j + 1) if j + 1 < NCHUNK else None
        pend[0].wait()
        pend[1].wait()
        b = j % 2
        off = base + j * CHUNK
        pltpu.sync_copy(urows_v.at[b], u_hbm.at[pl.ds(off, CHUNK)])
        pltpu.sync_copy(irows_v.at[b], v_hbm.at[pl.ds(off, CHUNK)])
        pend = nxt


def _sc_gather(uidx, iidx, user_emb2, item_emb2):
    k = pl.kernel(
        _gather_body,
        out_type=(
            jax.ShapeDtypeStruct((BATCH, ROW), jnp.float32),
            jax.ShapeDtypeStruct((BATCH, ROW), jnp.float32),
        ),
        mesh=_MESH,
        scratch_types=[
            pltpu.VMEM((NCHUNK, CHUNK), jnp.int32),
            pltpu.VMEM((NCHUNK, CHUNK), jnp.int32),
            pltpu.VMEM((2, CHUNK, ROW), jnp.float32),
            pltpu.VMEM((2, CHUNK, ROW), jnp.float32),
            pltpu.SemaphoreType.DMA,
        ],
    )
    return k(uidx, iidx, user_emb2, item_emb2)


BLK = 2048


def _mlp_body(u_ref, v_ref, up_ref, ip_ref, w1a_ref, w1b_ref, b1_ref,
              w2_ref, b2_ref, w3_ref, b3_ref, wp_ref, bp_ref, out_ref):
    f32 = jnp.float32
    u = jnp.where(up_ref[...] > 0, u_ref[:, EMBED_DIM:], u_ref[:, :EMBED_DIM])
    v = jnp.where(ip_ref[...] > 0, v_ref[:, EMBED_DIM:], v_ref[:, :EMBED_DIM])
    h = jnp.dot(u, w1a_ref[...], preferred_element_type=f32)
    h += jnp.dot(v, w1b_ref[...], preferred_element_type=f32)
    h = jnp.maximum(h + b1_ref[...], 0.0)
    h = jnp.maximum(jnp.dot(h, w2_ref[...], preferred_element_type=f32)
                    + b2_ref[...], 0.0)
    h = jnp.maximum(jnp.dot(h, w3_ref[...], preferred_element_type=f32)
                    + b3_ref[...], 0.0)
    p = jnp.sum(h * wp_ref[...], axis=1) + bp_ref[0, 0]
    out_ref[...] = jax.nn.sigmoid(p)


def _tc_mlp(u, v, upar, ipar, W1, b1, W2, b2, W3, b3, Wp, bp):
    w1a = W1[:, :EMBED_DIM].T
    w1b = W1[:, EMBED_DIM:].T
    w2 = W2.T
    w3 = W3.T
    grid = (BATCH // BLK,)
    full = lambda shape: pl.BlockSpec(shape, lambda i: (0,) * len(shape))
    return pl.pallas_call(
        _mlp_body,
        grid=grid,
        in_specs=[
            pl.BlockSpec((BLK, ROW), lambda i: (i, 0)),
            pl.BlockSpec((BLK, ROW), lambda i: (i, 0)),
            pl.BlockSpec((BLK, 1), lambda i: (i, 0)),
            pl.BlockSpec((BLK, 1), lambda i: (i, 0)),
            full((EMBED_DIM, 128)),
            full((EMBED_DIM, 128)),
            full((1, 128)),
            full((128, EMBED_DIM)),
            full((1, EMBED_DIM)),
            full((EMBED_DIM, 32)),
            full((1, 32)),
            full((1, 32)),
            full((1, 1)),
        ],
        out_specs=pl.BlockSpec((BLK,), lambda i: (i,)),
        out_shape=jax.ShapeDtypeStruct((BATCH,), jnp.float32),
    )(u, v, upar, ipar, w1a, w1b, b1.reshape(1, -1), w2, b2.reshape(1, -1),
      w3, b3.reshape(1, -1), Wp, bp.reshape(1, 1))


def kernel(user_indices, item_indices, user_emb, item_emb,
           W1, b1, W2, b2, W3, b3, Wp, bp):
    ui = user_indices.astype(jnp.int32)
    ii = item_indices.astype(jnp.int32)
    uidx = (ui // 2).reshape(NUM_TILES, NCHUNK, CHUNK)
    iidx = (ii // 2).reshape(NUM_TILES, NCHUNK, CHUNK)
    upar = (ui % 2).reshape(BATCH, 1)
    ipar = (ii % 2).reshape(BATCH, 1)
    uemb2 = user_emb.reshape(-1, ROW)
    iemb2 = item_emb.reshape(-1, ROW)
    u, v = _sc_gather(uidx, iidx, uemb2, iemb2)
    return _tc_mlp(u, v, upar, ipar, W1, b1, W2, b2, W3, b3, Wp, bp)


# speedup vs baseline: 3.4665x; 3.4665x over previous
import jax
import jax.numpy as jnp
from jax import lax
from jax.experimental import pallas as pl
from jax.experimental.pallas import tpu as pltpu
from jax.experimental.pallas import tpu_sc as plsc

BATCH = 16384
EMBED_DIM = 64
ROW = 2 * EMBED_DIM
NUM_CORES = 2
NUM_SUBCORES = 16
NUM_TILES = NUM_CORES * NUM_SUBCORES
B_PER_TILE = BATCH // NUM_TILES
CHUNK = 128
NCHUNK = B_PER_TILE // CHUNK

REPACK_C = 12800
PAIR_K = 512000

_MESH = plsc.VectorSubcoreMesh(core_axis_name="c", subcore_axis_name="s")


def _repack_body(a_ref, b_ref, out_ref):
    a = a_ref[...]
    b = b_ref[...]
    ab = jnp.concatenate([a, b], axis=0)
    eye = (jax.lax.broadcasted_iota(jnp.int32, (ROW, ROW), 0) ==
           jax.lax.broadcasted_iota(jnp.int32, (ROW, ROW), 1)).astype(jnp.float32)
    t = jax.lax.dot_general(ab, eye, (((0,), (0,)), ((), ())),
                            preferred_element_type=jnp.float32)
    out_ref[...] = pltpu.bitcast(t.astype(jnp.bfloat16), jnp.int32)


def _tc_repack(table_t):
    grid = (PAIR_K // REPACK_C,)
    return pl.pallas_call(
        _repack_body,
        grid=grid,
        in_specs=[
            pl.BlockSpec((EMBED_DIM, REPACK_C), lambda i: (0, i)),
            pl.BlockSpec((EMBED_DIM, REPACK_C),
                         lambda i: (0, jnp.minimum(i + PAIR_K // REPACK_C,
                                                   2 * PAIR_K // REPACK_C - 2))),
        ],
        out_specs=pl.BlockSpec((REPACK_C // 2, ROW), lambda i: (i, 0)),
        out_shape=jax.ShapeDtypeStruct((PAIR_K // 2, ROW), jnp.int32),
    )(table_t, table_t)


def _gather_body(uidx_hbm, iidx_hbm, uemb_hbm, iemb_hbm, u_hbm, v_hbm,
                 uidx_v, iidx_v, urows_v, irows_v, sem):
    wid = lax.axis_index("s") * NUM_CORES + lax.axis_index("c")
    base = wid * B_PER_TILE
    pltpu.sync_copy(uidx_hbm.at[wid], uidx_v)
    pltpu.sync_copy(iidx_hbm.at[wid], iidx_v)

    def start(j):
        b = j % 2
        return (
            pltpu.async_copy(uemb_hbm.at[uidx_v.at[j]], urows_v.at[b], sem),
            pltpu.async_copy(iemb_hbm.at[iidx_v.at[j]], irows_v.at[b], sem),
        )

    pend = start(0)
    for j in range(NCHUNK):
        nxt = start(j + 1) if j + 1 < NCHUNK else None
        pend[0].wait()
        pend[1].wait()
        b = j % 2
        off = base + j * CHUNK
        pltpu.sync_copy(urows_v.at[b], u_hbm.at[pl.ds(off, CHUNK)])
        pltpu.sync_copy(irows_v.at[b], v_hbm.at[pl.ds(off, CHUNK)])
        pend = nxt


def _sc_gather(uidx, iidx, uemb_p, iemb_p):
    k = pl.kernel(
        _gather_body,
        out_type=(
            jax.ShapeDtypeStruct((BATCH, ROW), jnp.int32),
            jax.ShapeDtypeStruct((BATCH, ROW), jnp.int32),
        ),
        mesh=_MESH,
        scratch_types=[
            pltpu.VMEM((NCHUNK, CHUNK), jnp.int32),
            pltpu.VMEM((NCHUNK, CHUNK), jnp.int32),
            pltpu.VMEM((2, CHUNK, ROW), jnp.int32),
            pltpu.VMEM((2, CHUNK, ROW), jnp.int32),
            pltpu.SemaphoreType.DMA,
        ],
    )
    return k(uidx, iidx, uemb_p, iemb_p)


BLK = 2048


def _unpack(xi, rp, par):
    bits = jnp.where(rp > 0, xi & jnp.int32(-65536), xi << 16)
    f = jax.lax.bitcast_convert_type(bits, jnp.float32)
    return jnp.where(par > 0, f[:, EMBED_DIM:], f[:, :EMBED_DIM])


def _mlp_body(u_ref, v_ref, urp_ref, up_ref, irp_ref, ip_ref,
              w1a_ref, w1b_ref, b1_ref, w2_ref, b2_ref,
              w3_ref, b3_ref, wp_ref, bp_ref, out_ref):
    f32 = jnp.float32
    u = _unpack(u_ref[...], urp_ref[...], up_ref[...])
    v = _unpack(v_ref[...], irp_ref[...], ip_ref[...])
    h = jnp.dot(u, w1a_ref[...], preferred_element_type=f32)
    h += jnp.dot(v, w1b_ref[...], preferred_element_type=f32)
    h = jnp.maximum(h + b1_ref[...], 0.0)
    h = jnp.maximum(jnp.dot(h, w2_ref[...], preferred_element_type=f32)
                    + b2_ref[...], 0.0)
    h = jnp.maximum(jnp.dot(h, w3_ref[...], preferred_element_type=f32)
                    + b3_ref[...], 0.0)
    p = jnp.sum(h * wp_ref[...], axis=1) + bp_ref[0, 0]
    out_ref[...] = jax.nn.sigmoid(p)


def _tc_mlp(u, v, urp, upar, irp, ipar, W1, b1, W2, b2, W3, b3, Wp, bp):
    w1a = W1[:, :EMBED_DIM].T
    w1b = W1[:, EMBED_DIM:].T
    w2 = W2.T
    w3 = W3.T
    grid = (BATCH // BLK,)
    full = lambda shape: pl.BlockSpec(shape, lambda i: (0,) * len(shape))
    par_spec = pl.BlockSpec((BLK, 1), lambda i: (i, 0))
    return pl.pallas_call(
        _mlp_body,
        grid=grid,
        in_specs=[
            pl.BlockSpec((BLK, ROW), lambda i: (i, 0)),
            pl.BlockSpec((BLK, ROW), lambda i: (i, 0)),
            par_spec, par_spec, par_spec, par_spec,
            full((EMBED_DIM, 128)),
            full((EMBED_DIM, 128)),
            full((1, 128)),
            full((128, EMBED_DIM)),
            full((1, EMBED_DIM)),
            full((EMBED_DIM, 32)),
            full((1, 32)),
            full((1, 32)),
            full((1, 1)),
        ],
        out_specs=pl.BlockSpec((BLK,), lambda i: (i,)),
        out_shape=jax.ShapeDtypeStruct((BATCH,), jnp.float32),
    )(u, v, urp, upar, irp, ipar, w1a, w1b, b1.reshape(1, -1),
      w2, b2.reshape(1, -1), w3, b3.reshape(1, -1), Wp, bp.reshape(1, 1))


def kernel(user_indices, item_indices, user_emb, item_emb,
           W1, b1, W2, b2, W3, b3, Wp, bp):
    ui = user_indices.astype(jnp.int32)
    ii = item_indices.astype(jnp.int32)
    ur = jnp.where(ui < PAIR_K, ui, ui - PAIR_K)
    ir = jnp.where(ii < PAIR_K, ii, ii - PAIR_K)
    uidx = (ur // 2).reshape(NUM_TILES, NCHUNK, CHUNK)
    iidx = (ir // 2).reshape(NUM_TILES, NCHUNK, CHUNK)
    urp = (ur % 2).reshape(BATCH, 1)
    irp = (ir % 2).reshape(BATCH, 1)
    upar = (ui >= PAIR_K).astype(jnp.int32).reshape(BATCH, 1)
    ipar = (ii >= PAIR_K).astype(jnp.int32).reshape(BATCH, 1)
    uemb_p = _tc_repack(user_emb.T)
    iemb_p = _tc_repack(item_emb.T)
    u, v = _sc_gather(uidx, iidx, uemb_p, iemb_p)
    return _tc_mlp(u, v, urp, upar, irp, ipar,
                   W1, b1, W2, b2, W3, b3, Wp, bp)
